# baseline (speedup 1.0000x reference)
import functools

import jax
import jax.numpy as jnp
from jax import lax
from jax.experimental import pallas as pl
from jax.experimental.pallas import tpu as pltpu
from jax.experimental.pallas import tpu_sc as plsc

N_NODES = 10000
N_EDGES = 320000
N_H = 128
N_G = 128
N_X = 16
N_U = 16
N_GRAPHS = 16

_NC = 2
_NS = 16
_NW = _NC * _NS
_EPW = N_EDGES // _NW
_CHUNK = 400

_PROJ_BLK = 1000
_MLP_BLK = 16000


def _proj_body(xh_ref, xg_ref, w1h_ref, w1g_ref, ph_ref, pg_ref):
    ph_ref[...] = jnp.dot(xh_ref[...], w1h_ref[...],
                          preferred_element_type=jnp.float32)
    pg_ref[...] = jnp.dot(xg_ref[...], w1g_ref[...],
                          preferred_element_type=jnp.float32)


def _project(x_h, x_g, w1h, w1g):
    grid = N_NODES // _PROJ_BLK
    return pl.pallas_call(
        _proj_body,
        grid=(grid,),
        in_specs=[
            pl.BlockSpec((_PROJ_BLK, N_H), lambda i: (i, 0)),
            pl.BlockSpec((_PROJ_BLK, N_G), lambda i: (i, 0)),
            pl.BlockSpec((N_H, N_X), lambda i: (0, 0)),
            pl.BlockSpec((N_G, N_X), lambda i: (0, 0)),
        ],
        out_specs=[
            pl.BlockSpec((_PROJ_BLK, N_X), lambda i: (i, 0)),
            pl.BlockSpec((_PROJ_BLK, N_X), lambda i: (i, 0)),
        ],
        out_shape=[
            jax.ShapeDtypeStruct((N_NODES, N_X), jnp.float32),
            jax.ShapeDtypeStruct((N_NODES, N_X), jnp.float32),
        ],
    )(x_h, x_g, w1h, w1g)


def _gather_body(ph_hbm, pg_hbm, ei_hbm, *rest):
    outs = rest[:N_X]
    (srcb, tgtb, h_bufs, g_bufs, ot_bufs,
     sem_a, sem_b, semo_a, semo_b) = rest[N_X:]
    wid = lax.axis_index("s") * _NC + lax.axis_index("c")
    base = wid * _EPW
    lane = lax.iota(jnp.int32, 16)
    fwd = {k: (lane + k) & 15 for k in (1, 2, 4, 8)}
    bwd = {k: (lane - k) & 15 for k in (1, 2, 4, 8)}
    keep = {k: (lane & k) == 0 for k in (1, 2, 4, 8)}
    sems = (sem_a, sem_b)
    semos = (semo_a, semo_b)
    nchunk = _EPW // _CHUNK

    pltpu.sync_copy(ei_hbm.at[0, pl.ds(base, _EPW)], srcb)
    pltpu.sync_copy(ei_hbm.at[1, pl.ds(base, _EPW)], tgtb)

    def issue(ci):
        pr = ci % 2
        cph = pltpu.async_copy(
            ph_hbm.at[srcb.at[pl.ds(ci * _CHUNK, _CHUNK)]], h_bufs[pr],
            sems[pr])
        cpg = pltpu.async_copy(
            pg_hbm.at[tgtb.at[pl.ds(ci * _CHUNK, _CHUNK)]], g_bufs[pr],
            sems[pr])
        return cph, cpg

    def butterfly(h_v, g_v, ot_v):
        def tile(t, c):
            rbase = t * 16
            vs = [h_v[rbase + i, :] + g_v[rbase + i, :] for i in range(16)]
            for k in (1, 2, 4, 8):
                nxt = list(vs)
                for i in range(16):
                    if i & k:
                        continue
                    p = i | k
                    a, b = vs[i], vs[p]
                    br = b.at[bwd[k]].get(mode="promise_in_bounds")
                    ar = a.at[fwd[k]].get(mode="promise_in_bounds")
                    nxt[i] = jnp.where(keep[k], a, br)
                    nxt[p] = jnp.where(keep[k], ar, b)
                vs = nxt
            for f in range(N_X):
                ot_v[pl.ds(f * _CHUNK + rbase, 16)] = vs[f]
            return c

        lax.fori_loop(0, _CHUNK // 16, tile, 0)

    pend_gather = issue(0)
    pend_out = [None, None]
    for ci in range(nchunk):
        pr = ci % 2
        if ci + 1 < nchunk:
            nxt_gather = issue(ci + 1)
        for cp in pend_gather:
            cp.wait()
        if pend_out[pr] is not None:
            for cp in pend_out[pr]:
                cp.wait()
        butterfly(h_bufs[pr], g_bufs[pr], ot_bufs[pr])
        off = base + ci * _CHUNK
        pend_out[pr] = [
            pltpu.async_copy(ot_bufs[pr].at[pl.ds(f * _CHUNK, _CHUNK)],
                             outs[f].at[pl.ds(off, _CHUNK)], semos[pr])
            for f in range(N_X)]
        if ci + 1 < nchunk:
            pend_gather = nxt_gather
    for pend in pend_out:
        if pend is not None:
            for cp in pend:
                cp.wait()


@functools.cache
def _gather_sum():
    buf = lambda shape, dt: pltpu.VMEM(shape, dt)
    return pl.kernel(
        _gather_body,
        out_type=[jax.ShapeDtypeStruct((N_EDGES,), jnp.float32)
                  for _ in range(N_X)],
        mesh=plsc.VectorSubcoreMesh(core_axis_name="c", subcore_axis_name="s",
                                    num_cores=_NC, num_subcores=_NS),
        compiler_params=pltpu.CompilerParams(use_tc_tiling_on_sc=False),
        scratch_types=[
            buf((_EPW,), jnp.int32),
            buf((_EPW,), jnp.int32),
            [buf((_CHUNK, N_X), jnp.float32) for _ in range(2)],
            [buf((_CHUNK, N_X), jnp.float32) for _ in range(2)],
            [buf((N_X * _CHUNK,), jnp.float32) for _ in range(2)],
            pltpu.SemaphoreType.DMA,
            pltpu.SemaphoreType.DMA,
            pltpu.SemaphoreType.DMA,
            pltpu.SemaphoreType.DMA,
        ],
    )


def _mlp_body(*refs):
    g_refs = refs[:N_X]
    (eat_ref, be_ref, w1xt_ref, ut_ref, w1ut_ref, b1_ref,
     w2t_ref, b2_ref, out_ref) = refs[N_X:]
    pu2t = jnp.dot(w1ut_ref[...], ut_ref[...],
                   preferred_element_type=jnp.float32) + b1_ref[...]
    be = be_ref[0]
    onehot = (jnp.broadcast_to(be, (N_GRAPHS, _MLP_BLK))
              == lax.broadcasted_iota(jnp.int32, (N_GRAPHS, _MLP_BLK), 0)
              ).astype(jnp.float32)
    gt = jnp.concatenate([r[0] for r in g_refs], axis=0)
    t = (gt
         + jnp.dot(w1xt_ref[...], eat_ref[...],
                   preferred_element_type=jnp.float32)
         + jnp.dot(pu2t, onehot, preferred_element_type=jnp.float32))
    h = jnp.where(t >= 0, t, 0.1 * t)
    out_ref[...] = jnp.dot(w2t_ref[...], h,
                           preferred_element_type=jnp.float32) + b2_ref[...]


def _mlp_tail(g_rows, eat, be3, w1xt, ut, w1ut, b1_col, w2t, b2_col):
    grid = N_EDGES // _MLP_BLK
    return pl.pallas_call(
        _mlp_body,
        grid=(grid,),
        in_specs=[
            pl.BlockSpec((1, 1, _MLP_BLK), lambda i: (i, 0, 0))
            for _ in range(N_X)
        ] + [
            pl.BlockSpec((N_X, _MLP_BLK), lambda i: (0, i)),
            pl.BlockSpec((1, 1, _MLP_BLK), lambda i: (i, 0, 0)),
            pl.BlockSpec((N_X, N_X), lambda i: (0, 0)),
            pl.BlockSpec((N_U, N_GRAPHS), lambda i: (0, 0)),
            pl.BlockSpec((N_X, N_U), lambda i: (0, 0)),
            pl.BlockSpec((N_X, 1), lambda i: (0, 0)),
            pl.BlockSpec((N_X, N_X), lambda i: (0, 0)),
            pl.BlockSpec((N_X, 1), lambda i: (0, 0)),
        ],
        out_specs=pl.BlockSpec((N_X, _MLP_BLK), lambda i: (0, i)),
        out_shape=jax.ShapeDtypeStruct((N_X, N_EDGES), jnp.float32),
    )(*g_rows, eat, be3, w1xt, ut, w1ut, b1_col, w2t, b2_col)


def kernel(x_h, x_g, edge_index, edge_attr, u, batch_e, W1, b1, W2, b2):
    ei = edge_index.astype(jnp.int32)
    be3 = batch_e.astype(jnp.int32).reshape(N_EDGES // _MLP_BLK, 1, _MLP_BLK)

    w1h = W1[:N_H]
    w1g = W1[N_H:N_H + N_G]
    w1xt = W1[N_H + N_G:N_H + N_G + N_X].T
    w1ut = W1[N_H + N_G + N_X:].T

    ph, pg = _project(x_h, x_g, w1h, w1g)
    g_rows = [r.reshape(N_EDGES // _MLP_BLK, 1, _MLP_BLK)
              for r in _gather_sum()(ph, pg, ei)]
    out_t = _mlp_tail(g_rows, edge_attr.T, be3, w1xt, u.T, w1ut,
                      b1.reshape(N_X, 1), W2.T, b2.reshape(N_X, 1))
    return out_t.T


# speedup vs baseline: 1.2139x; 1.2139x over previous
import functools

import jax
import jax.numpy as jnp
from jax import lax
from jax.experimental import pallas as pl
from jax.experimental.pallas import tpu as pltpu
from jax.experimental.pallas import tpu_sc as plsc

N_NODES = 10000
N_EDGES = 320000
N_H = 128
N_G = 128
N_X = 16
N_U = 16
N_GRAPHS = 16

_NC = 2
_NS = 16
_NW = _NC * _NS
_EPW = N_EDGES // _NW
_CHUNK = 400

_PROJ_BLK = 2000
_MLP_BLK = 32000


def _proj_body(xh_ref, xg_ref, w1h_ref, w1g_ref, ph_ref, pg_ref):
    ph_ref[...] = jnp.dot(xh_ref[...], w1h_ref[...],
                          preferred_element_type=jnp.float32)
    pg_ref[...] = jnp.dot(xg_ref[...], w1g_ref[...],
                          preferred_element_type=jnp.float32)


def _project(x_h, x_g, w1h, w1g):
    grid = N_NODES // _PROJ_BLK
    return pl.pallas_call(
        _proj_body,
        grid=(grid,),
        in_specs=[
            pl.BlockSpec((_PROJ_BLK, N_H), lambda i: (i, 0)),
            pl.BlockSpec((_PROJ_BLK, N_G), lambda i: (i, 0)),
            pl.BlockSpec((N_H, N_X), lambda i: (0, 0)),
            pl.BlockSpec((N_G, N_X), lambda i: (0, 0)),
        ],
        out_specs=[
            pl.BlockSpec((_PROJ_BLK, N_X), lambda i: (i, 0)),
            pl.BlockSpec((_PROJ_BLK, N_X), lambda i: (i, 0)),
        ],
        out_shape=[
            jax.ShapeDtypeStruct((N_NODES, N_X), jnp.float32),
            jax.ShapeDtypeStruct((N_NODES, N_X), jnp.float32),
        ],
    )(x_h, x_g, w1h, w1g)


def _gather_body(ph_hbm, pg_hbm, ei_hbm, out_hbm,
                 srcb, tgtb, h_bufs, g_bufs, ot_bufs,
                 sem_a, sem_b, semo_a, semo_b):
    wid = lax.axis_index("s") * _NC + lax.axis_index("c")
    base = wid * _EPW
    lane = lax.iota(jnp.int32, 16)
    fwd = {k: (lane + k) & 15 for k in (1, 2, 4, 8)}
    bwd = {k: (lane - k) & 15 for k in (1, 2, 4, 8)}
    keep = {k: (lane & k) == 0 for k in (1, 2, 4, 8)}
    sems = (sem_a, sem_b)
    semos = (semo_a, semo_b)
    nchunk = _EPW // _CHUNK

    pltpu.sync_copy(ei_hbm.at[0, pl.ds(base, _EPW)], srcb)
    pltpu.sync_copy(ei_hbm.at[1, pl.ds(base, _EPW)], tgtb)

    def issue(ci):
        pr = ci % 2
        cph = pltpu.async_copy(
            ph_hbm.at[srcb.at[pl.ds(ci * _CHUNK, _CHUNK)]], h_bufs[pr],
            sems[pr])
        cpg = pltpu.async_copy(
            pg_hbm.at[tgtb.at[pl.ds(ci * _CHUNK, _CHUNK)]], g_bufs[pr],
            sems[pr])
        return cph, cpg

    def butterfly(h_v, g_v, ot_v):
        def tile(t, c):
            rbase = t * 16
            vs = [h_v[rbase + i, :] + g_v[rbase + i, :] for i in range(16)]
            for k in (1, 2, 4, 8):
                nxt = list(vs)
                for i in range(16):
                    if i & k:
                        continue
                    p = i | k
                    a, b = vs[i], vs[p]
                    br = b.at[bwd[k]].get(mode="promise_in_bounds")
                    ar = a.at[fwd[k]].get(mode="promise_in_bounds")
                    nxt[i] = jnp.where(keep[k], a, br)
                    nxt[p] = jnp.where(keep[k], ar, b)
                vs = nxt
            for f in range(N_X):
                ot_v[pl.ds(f * _CHUNK + rbase, 16)] = vs[f]
            return c

        lax.fori_loop(0, _CHUNK // 16, tile, 0)

    pend_gather = issue(0)
    pend_out = [None, None]
    for ci in range(nchunk):
        pr = ci % 2
        if ci + 1 < nchunk:
            nxt_gather = issue(ci + 1)
        for cp in pend_gather:
            cp.wait()
        if pend_out[pr] is not None:
            for cp in pend_out[pr]:
                cp.wait()
        butterfly(h_bufs[pr], g_bufs[pr], ot_bufs[pr])
        off = base + ci * _CHUNK
        pend_out[pr] = [
            pltpu.async_copy(ot_bufs[pr].at[pl.ds(f * _CHUNK, _CHUNK)],
                             out_hbm.at[f, pl.ds(off, _CHUNK)], semos[pr])
            for f in range(N_X)]
        if ci + 1 < nchunk:
            pend_gather = nxt_gather
    for pend in pend_out:
        if pend is not None:
            for cp in pend:
                cp.wait()


@functools.cache
def _gather_sum():
    buf = lambda shape, dt: pltpu.VMEM(shape, dt)
    return pl.kernel(
        _gather_body,
        out_type=jax.ShapeDtypeStruct((N_X, N_EDGES), jnp.float32),
        mesh=plsc.VectorSubcoreMesh(core_axis_name="c", subcore_axis_name="s",
                                    num_cores=_NC, num_subcores=_NS),
        compiler_params=pltpu.CompilerParams(use_tc_tiling_on_sc=False),
        scratch_types=[
            buf((_EPW,), jnp.int32),
            buf((_EPW,), jnp.int32),
            [buf((_CHUNK, N_X), jnp.float32) for _ in range(2)],
            [buf((_CHUNK, N_X), jnp.float32) for _ in range(2)],
            [buf((N_X * _CHUNK,), jnp.float32) for _ in range(2)],
            pltpu.SemaphoreType.DMA,
            pltpu.SemaphoreType.DMA,
            pltpu.SemaphoreType.DMA,
            pltpu.SemaphoreType.DMA,
        ],
    )


def _mlp_body(gp_ref, eat_ref, be_ref, w1xt_ref, ut_ref, w1ut_ref, b1_ref,
              w2t_ref, b2_ref, out_ref):
    pu2t = jnp.dot(w1ut_ref[...], ut_ref[...],
                   preferred_element_type=jnp.float32) + b1_ref[...]
    be = be_ref[0]
    onehot = (jnp.broadcast_to(be, (N_GRAPHS, _MLP_BLK))
              == lax.broadcasted_iota(jnp.int32, (N_GRAPHS, _MLP_BLK), 0)
              ).astype(jnp.float32)
    t = (gp_ref[...]
         + jnp.dot(w1xt_ref[...], eat_ref[...],
                   preferred_element_type=jnp.float32)
         + jnp.dot(pu2t, onehot, preferred_element_type=jnp.float32))
    h = jnp.where(t >= 0, t, 0.1 * t)
    out_ref[...] = jnp.dot(w2t_ref[...], h,
                           preferred_element_type=jnp.float32) + b2_ref[...]


def _mlp_tail(gp, eat, be3, w1xt, ut, w1ut, b1_col, w2t, b2_col):
    grid = N_EDGES // _MLP_BLK
    return pl.pallas_call(
        _mlp_body,
        grid=(grid,),
        in_specs=[
            pl.BlockSpec((N_X, _MLP_BLK), lambda i: (0, i)),
            pl.BlockSpec((N_X, _MLP_BLK), lambda i: (0, i)),
            pl.BlockSpec((1, 1, _MLP_BLK), lambda i: (i, 0, 0)),
            pl.BlockSpec((N_X, N_X), lambda i: (0, 0)),
            pl.BlockSpec((N_U, N_GRAPHS), lambda i: (0, 0)),
            pl.BlockSpec((N_X, N_U), lambda i: (0, 0)),
            pl.BlockSpec((N_X, 1), lambda i: (0, 0)),
            pl.BlockSpec((N_X, N_X), lambda i: (0, 0)),
            pl.BlockSpec((N_X, 1), lambda i: (0, 0)),
        ],
        out_specs=pl.BlockSpec((N_X, _MLP_BLK), lambda i: (0, i)),
        out_shape=jax.ShapeDtypeStruct((N_X, N_EDGES), jnp.float32),
    )(gp, eat, be3, w1xt, ut, w1ut, b1_col, w2t, b2_col)


def kernel(x_h, x_g, edge_index, edge_attr, u, batch_e, W1, b1, W2, b2):
    ei = edge_index.astype(jnp.int32)
    be3 = batch_e.astype(jnp.int32).reshape(N_EDGES // _MLP_BLK, 1, _MLP_BLK)

    w1h = W1[:N_H]
    w1g = W1[N_H:N_H + N_G]
    w1xt = W1[N_H + N_G:N_H + N_G + N_X].T
    w1ut = W1[N_H + N_G + N_X:].T

    ph, pg = _project(x_h, x_g, w1h, w1g)
    gt = _gather_sum()(ph, pg, ei)
    out_t = _mlp_tail(gt, edge_attr.T, be3, w1xt, u.T, w1ut,
                      b1.reshape(N_X, 1), W2.T, b2.reshape(N_X, 1))
    return out_t.T
